# baseline (speedup 1.0000x reference)
import functools

import jax
import jax.numpy as jnp
from jax import lax
from jax.experimental import pallas as pl
from jax.experimental.pallas import tpu as pltpu
from jax.experimental.pallas import tpu_sc as plsc

N_FACTORS = 64
BATCH = 16384
NUM_CORES = 2
NUM_SUBCORES = 16
NW = NUM_CORES * NUM_SUBCORES
BPW = BATCH // NW
CH = 128
LANES = 16
ROW_UNROLL = 16

_mesh = plsc.VectorSubcoreMesh(
    core_axis_name="c", subcore_axis_name="s",
    num_cores=NUM_CORES, num_subcores=NUM_SUBCORES)


@functools.partial(
    pl.kernel,
    out_type=jax.ShapeDtypeStruct((BATCH,), jnp.float32),
    mesh=_mesh,
    compiler_params=pltpu.CompilerParams(needs_layout_passes=False),
    scratch_types=[
        pltpu.VMEM((BPW,), jnp.int32),
        pltpu.VMEM((BPW,), jnp.int32),
        pltpu.VMEM((CH, N_FACTORS), jnp.float32),
        pltpu.VMEM((CH, N_FACTORS), jnp.float32),
        pltpu.VMEM((CH, 1), jnp.float32),
        pltpu.VMEM((CH, 1), jnp.float32),
        pltpu.VMEM((BPW,), jnp.float32),
        pltpu.SemaphoreType.DMA,
    ],
)
def _mf_sc(u_idx_hbm, i_idx_hbm, u_emb_hbm, i_emb_hbm, ub_hbm, ib_hbm,
           out_hbm, uidx_v, iidx_v, ue_v, ie_v, ub_t, ib_t,
           out_v, sem_rows):
    wid = lax.axis_index("s") * NUM_CORES + lax.axis_index("c")
    base = wid * BPW

    pltpu.sync_copy(u_idx_hbm.at[pl.ds(base, BPW)], uidx_v)
    pltpu.sync_copy(i_idx_hbm.at[pl.ds(base, BPW)], iidx_v)

    lane = lax.iota(jnp.int32, LANES)
    zero16 = jnp.zeros((LANES,), jnp.int32)

    def run_pass(p, carry):
        p0 = p * CH

        def fire_chunk(c, carry2):
            i0 = c * ROW_UNROLL
            uvec = uidx_v[pl.ds(p0 + i0, ROW_UNROLL)]
            ivec = iidx_v[pl.ds(p0 + i0, ROW_UNROLL)]
            for j in range(ROW_UNROLL):
                i = i0 + j
                ru = uvec[j]
                ri = ivec[j]
                pltpu.make_async_copy(
                    u_emb_hbm.at[pl.ds(ru, 1), :], ue_v.at[pl.ds(i, 1), :],
                    sem_rows).start()
                pltpu.make_async_copy(
                    i_emb_hbm.at[pl.ds(ri, 1), :], ie_v.at[pl.ds(i, 1), :],
                    sem_rows).start()
                pltpu.make_async_copy(
                    ub_hbm.at[pl.ds(ru, 1), :], ub_t.at[pl.ds(i, 1), :],
                    sem_rows).start()
                pltpu.make_async_copy(
                    ib_hbm.at[pl.ds(ri, 1), :], ib_t.at[pl.ds(i, 1), :],
                    sem_rows).start()
            return carry2

        lax.fori_loop(0, CH // ROW_UNROLL, fire_chunk, 0)
        pltpu.make_async_copy(
            u_emb_hbm.at[pl.ds(0, CH), :], ue_v, sem_rows).wait()
        pltpu.make_async_copy(
            i_emb_hbm.at[pl.ds(0, CH), :], ie_v, sem_rows).wait()
        pltpu.make_async_copy(
            ub_hbm.at[pl.ds(0, CH), :], ub_t, sem_rows).wait()
        pltpu.make_async_copy(
            ib_hbm.at[pl.ds(0, CH), :], ib_t, sem_rows).wait()

        def group_body(g, carry2):
            r0 = g * LANES
            rows = r0 + lane
            o0 = p0 + r0
            acc = (plsc.load_gather(ub_t, [rows, zero16])
                   + plsc.load_gather(ib_t, [rows, zero16]))
            for f in range(N_FACTORS):
                fv = jnp.full((LANES,), f, jnp.int32)
                a = plsc.load_gather(ue_v, [rows, fv])
                b = plsc.load_gather(ie_v, [rows, fv])
                acc = acc + a * b
            out_v[pl.ds(o0, LANES)] = acc
            return carry2

        lax.fori_loop(0, CH // LANES, group_body, 0)
        return carry

    lax.fori_loop(0, BPW // CH, run_pass, 0)

    pltpu.sync_copy(out_v, out_hbm.at[pl.ds(base, BPW)])


def kernel(u_idx, i_idx, u_emb, i_emb, u_bias, i_bias):
    return _mf_sc(u_idx, i_idx, u_emb, i_emb, u_bias, i_bias)


# speedup vs baseline: 1.5981x; 1.5981x over previous
import functools

import jax
import jax.numpy as jnp
from jax import lax
from jax.experimental import pallas as pl
from jax.experimental.pallas import tpu as pltpu
from jax.experimental.pallas import tpu_sc as plsc

N_FACTORS = 64
BATCH = 16384
NUM_CORES = 2
NUM_SUBCORES = 16
NW = NUM_CORES * NUM_SUBCORES
BPW = BATCH // NW
CH = 256
CHUNK = 128
LANES = 16
ROW_UNROLL = 16

_mesh = plsc.VectorSubcoreMesh(
    core_axis_name="c", subcore_axis_name="s",
    num_cores=NUM_CORES, num_subcores=NUM_SUBCORES)


@functools.partial(
    pl.kernel,
    out_type=jax.ShapeDtypeStruct((BATCH,), jnp.float32),
    mesh=_mesh,
    compiler_params=pltpu.CompilerParams(needs_layout_passes=False),
    scratch_types=[
        pltpu.VMEM((BPW,), jnp.int32),
        pltpu.VMEM((BPW,), jnp.int32),
        pltpu.VMEM((CH, N_FACTORS), jnp.float32),
        pltpu.VMEM((CH, N_FACTORS), jnp.float32),
        pltpu.VMEM((BPW,), jnp.float32),
        pltpu.VMEM((BPW,), jnp.float32),
        pltpu.VMEM((BPW,), jnp.float32),
        pltpu.SemaphoreType.DMA,
        pltpu.SemaphoreType.DMA,
    ],
)
def _mf_sc(u_idx_hbm, i_idx_hbm, u_emb_hbm, i_emb_hbm, ub_hbm, ib_hbm,
           out_hbm, uidx_v, iidx_v, ue_v, ie_v, ub_v, ib_v,
           out_v, sem_rows, sem_bias):
    wid = lax.axis_index("s") * NUM_CORES + lax.axis_index("c")
    base = wid * BPW

    pltpu.sync_copy(u_idx_hbm.at[pl.ds(base, BPW)], uidx_v)
    pltpu.sync_copy(i_idx_hbm.at[pl.ds(base, BPW)], iidx_v)

    bias_copies = []
    for c in range(BPW // CHUNK):
        sl = pl.ds(c * CHUNK, CHUNK)
        bias_copies.append(pltpu.async_copy(
            ub_hbm.at[0].at[uidx_v.at[sl]], ub_v.at[sl], sem_bias))
        bias_copies.append(pltpu.async_copy(
            ib_hbm.at[0].at[iidx_v.at[sl]], ib_v.at[sl], sem_bias))

    lane = lax.iota(jnp.int32, LANES)

    def run_pass(p, carry):
        p0 = p * CH

        def fire_chunk(c, carry2):
            i0 = c * ROW_UNROLL
            uvec = uidx_v[pl.ds(p0 + i0, ROW_UNROLL)]
            ivec = iidx_v[pl.ds(p0 + i0, ROW_UNROLL)]
            for j in range(ROW_UNROLL):
                i = i0 + j
                ru = uvec[j]
                ri = ivec[j]
                pltpu.make_async_copy(
                    u_emb_hbm.at[pl.ds(ru, 1), :], ue_v.at[pl.ds(i, 1), :],
                    sem_rows).start()
                pltpu.make_async_copy(
                    i_emb_hbm.at[pl.ds(ri, 1), :], ie_v.at[pl.ds(i, 1), :],
                    sem_rows).start()
            return carry2

        lax.fori_loop(0, CH // ROW_UNROLL, fire_chunk, 0)
        pltpu.make_async_copy(
            u_emb_hbm.at[pl.ds(0, CH), :], ue_v, sem_rows).wait()
        pltpu.make_async_copy(
            i_emb_hbm.at[pl.ds(0, CH), :], ie_v, sem_rows).wait()

        def group_body(g, carry2):
            r0 = g * LANES
            rows = r0 + lane
            o0 = p0 + r0
            acc = ub_v[pl.ds(o0, LANES)] + ib_v[pl.ds(o0, LANES)]
            for f in range(N_FACTORS):
                fv = jnp.full((LANES,), f, jnp.int32)
                a = plsc.load_gather(ue_v, [rows, fv])
                b = plsc.load_gather(ie_v, [rows, fv])
                acc = acc + a * b
            out_v[pl.ds(o0, LANES)] = acc
            return carry2

        lax.fori_loop(0, CH // LANES, group_body, 0)
        return carry

    for cp in bias_copies:
        cp.wait()
    lax.fori_loop(0, BPW // CH, run_pass, 0)

    pltpu.sync_copy(out_v, out_hbm.at[pl.ds(base, BPW)])


def kernel(u_idx, i_idx, u_emb, i_emb, u_bias, i_bias):
    return _mf_sc(u_idx, i_idx, u_emb, i_emb, u_bias.T, i_bias.T)
